# baseline (speedup 1.0000x reference)
import functools

import jax
import jax.numpy as jnp
from jax import lax
from jax.experimental import pallas as pl
from jax.experimental.pallas import tpu as pltpu
from jax.experimental.pallas import tpu_sc as plsc

_B, _L, _D = 64, 576, 768
_NC, _NS = 2, 16
_G, _J = 4, 8
_BPG = _B // _G
_RPW = _L // _J
_CHR = 24
_NCH = _RPW // _CHR
_NVC = _D // 16
_NBUF = 3


def _sc_body(in_hbm, pos_hbm, cls_hbm, out_hbm,
             pos_v, buf_v, cls_v, tmp_v, sem_in, sem_out):
    wid = lax.axis_index("s") * _NC + lax.axis_index("c")
    g = wid // _J
    j = wid % _J
    b0 = g * _BPG

    def chunk_body(ch, carry):
        r0 = j * _RPW + ch * _CHR
        pltpu.sync_copy(pos_hbm.at[pl.ds(r0, _CHR)], pos_v)

        def in_cp(bb, k):
            return pltpu.make_async_copy(
                in_hbm.at[bb, pl.ds(r0, _CHR)], buf_v.at[k], sem_in.at[k])

        def out_cp(bb, k):
            return pltpu.make_async_copy(
                buf_v.at[k], out_hbm.at[bb, pl.ds(r0, _CHR)], sem_out.at[k])

        in_cp(b0, 0).start()

        def batch_body(b, carry2):
            k = lax.rem(b, _NBUF)
            kn = lax.rem(b + 1, _NBUF)
            bb = b0 + b
            in_cp(bb, k).wait()

            @pl.when(b >= 2)
            def _():
                out_cp(bb, kn).wait()

            @pl.when(b + 1 < _BPG)
            def _():
                in_cp(bb + 1, kn).start()

            def row_body(r, carry3):
                for c in range(_NVC):
                    off = c * 16
                    buf_v[k, r, pl.ds(off, 16)] = (
                        buf_v[k, r, pl.ds(off, 16)] + pos_v[r, pl.ds(off, 16)]
                    )
                return carry3

            lax.fori_loop(0, _CHR, row_body, carry2)
            out_cp(bb, k).start()
            return carry2

        lax.fori_loop(0, _BPG, batch_body, carry)
        out_cp(b0 + _BPG - 2, (_BPG - 2) % _NBUF).wait()
        out_cp(b0 + _BPG - 1, (_BPG - 1) % _NBUF).wait()
        return carry

    lax.fori_loop(0, _NCH, chunk_body, 0)

    @pl.when(j == 0)
    def _():
        pltpu.sync_copy(cls_hbm.at[0], cls_v)
        pltpu.sync_copy(pos_hbm.at[pl.ds(_L, 1)], tmp_v)

        for c in range(_NVC):
            off = c * 16
            cls_v[0, pl.ds(off, 16)] = (
                cls_v[0, pl.ds(off, 16)] + tmp_v[0, pl.ds(off, 16)]
            )

        def cls_batch(b, carry):
            pltpu.sync_copy(cls_v, out_hbm.at[b0 + b, pl.ds(_L, 1)])
            return carry

        lax.fori_loop(0, _BPG, cls_batch, 0)


@functools.partial(
    pl.kernel,
    mesh=plsc.VectorSubcoreMesh(core_axis_name="c", subcore_axis_name="s"),
    out_type=jax.ShapeDtypeStruct((_B, _L + 1, _D), jnp.float32),
    scratch_types=[
        pltpu.VMEM((_CHR, _D), jnp.float32),
        pltpu.VMEM((_NBUF, _CHR, _D), jnp.float32),
        pltpu.VMEM((1, _D), jnp.float32),
        pltpu.VMEM((1, _D), jnp.float32),
        pltpu.SemaphoreType.DMA((_NBUF,)),
        pltpu.SemaphoreType.DMA((_NBUF,)),
    ],
)
def _sc_kernel(in_hbm, pos_hbm, cls_hbm, out_hbm,
               pos_v, buf_v, cls_v, tmp_v, sem_in, sem_out):
    _sc_body(in_hbm, pos_hbm, cls_hbm, out_hbm,
             pos_v, buf_v, cls_v, tmp_v, sem_in, sem_out)


def kernel(inputs, pos_table, class_token):
    return _sc_kernel(inputs, pos_table, class_token)


# speedup vs baseline: 1.4816x; 1.4816x over previous
import functools

import jax
import jax.numpy as jnp
from jax import lax
from jax.experimental import pallas as pl
from jax.experimental.pallas import tpu as pltpu
from jax.experimental.pallas import tpu_sc as plsc

_B, _L, _D = 64, 576, 768
_NC, _NS = 2, 16
_G, _J = 4, 8
_BPG = _B // _G
_RPW = _L // _J
_CHR = 24
_NCH = _RPW // _CHR
_NVC = _D // 16
_NBUF = 4


def _sc_body(in_hbm, pos_hbm, cls_hbm, out_hbm,
             pos_v, buf_v, cls_v, tmp_v, sem_in, sem_out):
    wid = lax.axis_index("s") * _NC + lax.axis_index("c")
    g = wid // _J
    j = wid % _J
    b0 = g * _BPG

    def chunk_body(ch, carry):
        r0 = j * _RPW + ch * _CHR
        pltpu.sync_copy(pos_hbm.at[pl.ds(r0, _CHR)], pos_v)

        def in_cp(bb, k):
            return pltpu.make_async_copy(
                in_hbm.at[bb, pl.ds(r0, _CHR)], buf_v.at[k], sem_in.at[k])

        def out_cp(bb, k):
            return pltpu.make_async_copy(
                buf_v.at[k], out_hbm.at[bb, pl.ds(r0, _CHR)], sem_out.at[k])

        in_cp(b0, 0).start()

        for b in range(_BPG):
            k = b % _NBUF
            kn = (b + 1) % _NBUF
            bb = b0 + b
            in_cp(bb, k).wait()
            if b >= _NBUF - 1:
                out_cp(b0 + b - (_NBUF - 1), kn).wait()
            if b + 1 < _BPG:
                in_cp(bb + 1, kn).start()

            def row_body(r, carry3, k=k):
                for c in range(_NVC):
                    off = c * 16
                    buf_v[k, r, pl.ds(off, 16)] = (
                        buf_v[k, r, pl.ds(off, 16)] + pos_v[r, pl.ds(off, 16)]
                    )
                return carry3

            lax.fori_loop(0, _CHR, row_body, 0)
            out_cp(bb, k).start()

        for b in range(_BPG - (_NBUF - 1), _BPG):
            out_cp(b0 + b, b % _NBUF).wait()
        return carry

    lax.fori_loop(0, _NCH, chunk_body, 0)

    @pl.when(j == 0)
    def _():
        pltpu.sync_copy(cls_hbm.at[0], cls_v)
        pltpu.sync_copy(pos_hbm.at[pl.ds(_L, 1)], tmp_v)

        for c in range(_NVC):
            off = c * 16
            cls_v[0, pl.ds(off, 16)] = (
                cls_v[0, pl.ds(off, 16)] + tmp_v[0, pl.ds(off, 16)]
            )

        def cls_batch(b, carry):
            pltpu.sync_copy(cls_v, out_hbm.at[b0 + b, pl.ds(_L, 1)])
            return carry

        lax.fori_loop(0, _BPG, cls_batch, 0)


@functools.partial(
    pl.kernel,
    mesh=plsc.VectorSubcoreMesh(core_axis_name="c", subcore_axis_name="s"),
    out_type=jax.ShapeDtypeStruct((_B, _L + 1, _D), jnp.float32),
    scratch_types=[
        pltpu.VMEM((_CHR, _D), jnp.float32),
        pltpu.VMEM((_NBUF, _CHR, _D), jnp.float32),
        pltpu.VMEM((1, _D), jnp.float32),
        pltpu.VMEM((1, _D), jnp.float32),
        pltpu.SemaphoreType.DMA((_NBUF,)),
        pltpu.SemaphoreType.DMA((_NBUF,)),
    ],
)
def _sc_kernel(in_hbm, pos_hbm, cls_hbm, out_hbm,
               pos_v, buf_v, cls_v, tmp_v, sem_in, sem_out):
    _sc_body(in_hbm, pos_hbm, cls_hbm, out_hbm,
             pos_v, buf_v, cls_v, tmp_v, sem_in, sem_out)


def kernel(inputs, pos_table, class_token):
    return _sc_kernel(inputs, pos_table, class_token)


# speedup vs baseline: 1.6589x; 1.1197x over previous
import functools

import jax
import jax.numpy as jnp
from jax import lax
from jax.experimental import pallas as pl
from jax.experimental.pallas import tpu as pltpu
from jax.experimental.pallas import tpu_sc as plsc

_B, _L, _D = 64, 576, 768
_NC, _NS = 2, 16
_G, _J = 4, 8
_BPG = _B // _G
_RPW = _L // _J
_CHR = 8
_NCH = _RPW // _CHR
_NVC = _D // 16
_QB = 4
_NQ = _BPG // _QB
_NRING = 3


def _sc_body(in_hbm, pos_hbm, cls_hbm, out_hbm,
             pos_v, buf_v, cls_v, tmp_v, sem_in, sem_out):
    wid = lax.axis_index("s") * _NC + lax.axis_index("c")
    g = wid // _J
    j = wid % _J
    b0 = g * _BPG

    def chunk_body(ch, carry):
        r0 = j * _RPW + ch * _CHR
        pltpu.sync_copy(pos_hbm.at[pl.ds(r0, _CHR)], pos_v)

        def in_q(q, t):
            return [
                pltpu.make_async_copy(
                    in_hbm.at[b0 + q * _QB + k, pl.ds(r0, _CHR)],
                    buf_v.at[t, k], sem_in.at[t])
                for k in range(_QB)
            ]

        def out_q(q, t):
            return [
                pltpu.make_async_copy(
                    buf_v.at[t, k],
                    out_hbm.at[b0 + q * _QB + k, pl.ds(r0, _CHR)],
                    sem_out.at[t])
                for k in range(_QB)
            ]

        for cp in in_q(0, 0):
            cp.start()

        for q in range(_NQ):
            t = q % _NRING
            tn = (q + 1) % _NRING
            for cp in in_q(q, t):
                cp.wait()
            if q >= 2:
                for cp in out_q(q - 2, tn):
                    cp.wait()
            if q + 1 < _NQ:
                for cp in in_q(q + 1, tn):
                    cp.start()

            def row_body(r, carry3, t=t):
                for c in range(_NVC):
                    off = c * 16
                    p = pos_v[r, pl.ds(off, 16)]
                    for k in range(_QB):
                        buf_v[t, k, r, pl.ds(off, 16)] = (
                            buf_v[t, k, r, pl.ds(off, 16)] + p
                        )
                return carry3

            lax.fori_loop(0, _CHR, row_body, 0)
            for cp in out_q(q, t):
                cp.start()

        for q in range(_NQ - 2, _NQ):
            for cp in out_q(q, q % _NRING):
                cp.wait()
        return carry

    lax.fori_loop(0, _NCH, chunk_body, 0)

    @pl.when(j == 0)
    def _():
        pltpu.sync_copy(cls_hbm.at[0], cls_v)
        pltpu.sync_copy(pos_hbm.at[pl.ds(_L, 1)], tmp_v)

        for c in range(_NVC):
            off = c * 16
            cls_v[0, pl.ds(off, 16)] = (
                cls_v[0, pl.ds(off, 16)] + tmp_v[0, pl.ds(off, 16)]
            )

        def cls_batch(b, carry):
            pltpu.sync_copy(cls_v, out_hbm.at[b0 + b, pl.ds(_L, 1)])
            return carry

        lax.fori_loop(0, _BPG, cls_batch, 0)


@functools.partial(
    pl.kernel,
    mesh=plsc.VectorSubcoreMesh(core_axis_name="c", subcore_axis_name="s"),
    out_type=jax.ShapeDtypeStruct((_B, _L + 1, _D), jnp.float32),
    scratch_types=[
        pltpu.VMEM((_CHR, _D), jnp.float32),
        pltpu.VMEM((_NRING, _QB, _CHR, _D), jnp.float32),
        pltpu.VMEM((1, _D), jnp.float32),
        pltpu.VMEM((1, _D), jnp.float32),
        pltpu.SemaphoreType.DMA((_NRING,)),
        pltpu.SemaphoreType.DMA((_NRING,)),
    ],
)
def _sc_kernel(in_hbm, pos_hbm, cls_hbm, out_hbm,
               pos_v, buf_v, cls_v, tmp_v, sem_in, sem_out):
    _sc_body(in_hbm, pos_hbm, cls_hbm, out_hbm,
             pos_v, buf_v, cls_v, tmp_v, sem_in, sem_out)


def kernel(inputs, pos_table, class_token):
    return _sc_kernel(inputs, pos_table, class_token)
